# baseline (speedup 1.0000x reference)
import functools

import jax
import jax.numpy as jnp
import numpy as np
from jax import lax
from jax.experimental import pallas as pl
from jax.experimental.pallas import tpu as pltpu
from jax.experimental.pallas import tpu_sc as plsc

D_MODEL = 1024
MAX_LEN = 2048
BATCH = 4
SEQ = 2048
SCALE = 32.0

L = 16
NC = 2
NS = 16
NW = NC * NS
S_PER_W = SEQ // NW
CHUNK = 16
STEPS = S_PER_W // CHUNK
NSTEP = BATCH * STEPS
PAIRS = D_MODEL // (2 * L)
PE_W = D_MODEL // 2


def _positional_encoding_packed():
    d = D_MODEL / 2
    pos = np.arange(MAX_LEN)[:, np.newaxis]
    dims = np.arange(d)[np.newaxis, :] / d
    frequency = pos * (1.0 / 10000 ** dims)
    pe = np.concatenate([np.sin(frequency), np.cos(frequency)], axis=-1)
    pe = pe.astype(np.float32).reshape(MAX_LEN, D_MODEL // 32, 2, 16)
    u = pe.view(np.uint32)
    bits = ((u + 0x7FFF + ((u >> 16) & 1)) >> 16).astype(np.uint32)
    words = bits[:, :, 0, :] | (bits[:, :, 1, :] << 16)
    words = words.astype(np.uint32).view(np.int32)
    return jnp.asarray(words.reshape(MAX_LEN * PE_W))


def _sc_embed(x, pe, table):
    mesh = plsc.VectorSubcoreMesh(core_axis_name="c", subcore_axis_name="s")

    @functools.partial(
        pl.kernel,
        mesh=mesh,
        out_type=jax.ShapeDtypeStruct((BATCH, SEQ, D_MODEL), jnp.float32),
        scratch_types=[
            pltpu.VMEM((BATCH * S_PER_W,), jnp.int32),
            pltpu.VMEM((S_PER_W * PE_W,), jnp.int32),
            pltpu.VMEM((2, CHUNK, D_MODEL), jnp.float32),
            pltpu.VMEM((2, CHUNK, D_MODEL), jnp.float32),
            pltpu.SemaphoreType.DMA((2,)),
            pltpu.SemaphoreType.DMA((2,)),
            pltpu.SemaphoreType.DMA,
        ],
    )
    def k(x_hbm, pe_hbm, table_hbm, out_hbm, idx_v, pe_v,
          gbufs, obufs, g_sems, st_sems, pe_sem):

        wid = lax.axis_index("s") * NC + lax.axis_index("c")
        s0 = wid * S_PER_W

        pe_copy = pltpu.async_copy(
            pe_hbm.at[pl.ds(s0 * PE_W, S_PER_W * PE_W)], pe_v, pe_sem
        )
        for b in range(BATCH):
            pltpu.sync_copy(
                x_hbm.at[pl.ds(b * SEQ + s0, S_PER_W)],
                idx_v.at[pl.ds(b * S_PER_W, S_PER_W)],
            )

        def fire_gather(s, ph):
            idx_slice = idx_v.at[pl.ds(s * CHUNK, CHUNK)]
            return pltpu.async_copy(
                table_hbm.at[idx_slice], gbufs.at[ph], g_sems.at[ph]
            )

        fire_gather(0, 0)
        fire_gather(1, 1)
        pe_copy.wait()

        def compute(gbuf, obuf, c):
            def row_body(r, _):
                pe_off = (c * CHUNK + r) * PE_W
                PF = 2
                ws, gas, gbs = [], [], []
                for j in range(PAIRS + PF):
                    if j < PAIRS:
                        col = j * 2 * L
                        ws.append(pe_v[pl.ds(pe_off + j * L, L)])
                        gas.append(gbuf[r, pl.ds(col, L)])
                        gbs.append(gbuf[r, pl.ds(col + L, L)])
                    if j >= PF:
                        kk = j - PF
                        colk = kk * 2 * L
                        w = ws[kk]
                        pa = lax.bitcast_convert_type(w << 16, jnp.float32)
                        pb = lax.bitcast_convert_type(
                            (w >> 16) << 16, jnp.float32)
                        obuf[r, pl.ds(colk, L)] = gas[kk] * SCALE + pa
                        obuf[r, pl.ds(colk + L, L)] = gbs[kk] * SCALE + pb
                return 0

            lax.fori_loop(0, CHUNK, row_body, 0)

        def outer(s, _):
            ph = lax.rem(s, 2)
            b = s // STEPS
            c = s % STEPS
            gbuf, obuf = gbufs.at[ph], obufs.at[ph]
            pltpu.make_async_copy(
                table_hbm.at[pl.ds(0, CHUNK)], gbuf, g_sems.at[ph]
            ).wait()

            @pl.when(s >= 2)
            def _():
                pltpu.make_async_copy(
                    obuf, out_hbm.at[0, pl.ds(0, CHUNK), :], st_sems.at[ph]
                ).wait()

            compute(gbuf, obuf, c)
            pltpu.async_copy(
                obuf,
                out_hbm.at[b, pl.ds(s0 + c * CHUNK, CHUNK), :],
                st_sems.at[ph],
            )

            @pl.when(s < NSTEP - 2)
            def _():
                fire_gather(s + 2, ph)

            return 0

        lax.fori_loop(0, NSTEP, outer, 0)

        for ph in range(2):
            pltpu.make_async_copy(
                obufs.at[ph], out_hbm.at[0, pl.ds(0, CHUNK), :], st_sems.at[ph]
            ).wait()

    return k(x, pe, table)


def kernel(x, table):
    pe = _positional_encoding_packed()
    return _sc_embed(x.reshape(-1).astype(jnp.int32), pe, table)


# speedup vs baseline: 1.0083x; 1.0083x over previous
import functools

import jax
import jax.numpy as jnp
import numpy as np
from jax import lax
from jax.experimental import pallas as pl
from jax.experimental.pallas import tpu as pltpu
from jax.experimental.pallas import tpu_sc as plsc

D_MODEL = 1024
MAX_LEN = 2048
BATCH = 4
SEQ = 2048
SCALE = 32.0

L = 16
NC = 2
NS = 16
NW = NC * NS
S_PER_W = SEQ // NW
CHUNK = 16
STEPS = S_PER_W // CHUNK
NSTEP = BATCH * STEPS
PAIRS = D_MODEL // (2 * L)
PE_W = D_MODEL // 2


def _positional_encoding_packed():
    d = D_MODEL / 2
    pos = np.arange(MAX_LEN)[:, np.newaxis]
    dims = np.arange(d)[np.newaxis, :] / d
    frequency = pos * (1.0 / 10000 ** dims)
    pe = np.concatenate([np.sin(frequency), np.cos(frequency)], axis=-1)
    pe = pe.astype(np.float32).reshape(MAX_LEN, D_MODEL // 32, 2, 16)
    u = pe.view(np.uint32)
    bits = ((u + 0x7FFF + ((u >> 16) & 1)) >> 16).astype(np.uint32)
    words = bits[:, :, 0, :] | (bits[:, :, 1, :] << 16)
    words = words.astype(np.uint32).view(np.int32)
    return jnp.asarray(words.reshape(MAX_LEN * PE_W))


def _sc_embed(x, pe, table):
    mesh = plsc.VectorSubcoreMesh(core_axis_name="c", subcore_axis_name="s")

    @functools.partial(
        pl.kernel,
        mesh=mesh,
        out_type=jax.ShapeDtypeStruct((BATCH, SEQ, D_MODEL), jnp.float32),
        scratch_types=[
            pltpu.VMEM((BATCH * S_PER_W,), jnp.int32),
            pltpu.VMEM((S_PER_W * PE_W,), jnp.int32),
            pltpu.VMEM((CHUNK, D_MODEL), jnp.float32),
            pltpu.VMEM((CHUNK, D_MODEL), jnp.float32),
            pltpu.VMEM((CHUNK, D_MODEL), jnp.float32),
            pltpu.VMEM((CHUNK, D_MODEL), jnp.float32),
            pltpu.SemaphoreType.DMA,
            pltpu.SemaphoreType.DMA,
            pltpu.SemaphoreType.DMA,
            pltpu.SemaphoreType.DMA,
            pltpu.SemaphoreType.DMA,
        ],
    )
    def k(x_hbm, pe_hbm, table_hbm, out_hbm, idx_v, pe_v,
          gbuf0, gbuf1, obuf0, obuf1, g_sem0, g_sem1, st_sem0, st_sem1,
          pe_sem):
        gbufs = (gbuf0, gbuf1)
        obufs = (obuf0, obuf1)
        g_sems = (g_sem0, g_sem1)
        st_sems = (st_sem0, st_sem1)

        wid = lax.axis_index("s") * NC + lax.axis_index("c")
        s0 = wid * S_PER_W

        pe_copy = pltpu.async_copy(
            pe_hbm.at[pl.ds(s0 * PE_W, S_PER_W * PE_W)], pe_v, pe_sem
        )
        for b in range(BATCH):
            pltpu.sync_copy(
                x_hbm.at[pl.ds(b * SEQ + s0, S_PER_W)],
                idx_v.at[pl.ds(b * S_PER_W, S_PER_W)],
            )

        def fire_gather(s, ph):
            idx_slice = idx_v.at[pl.ds(s * CHUNK, CHUNK)]
            return pltpu.async_copy(
                table_hbm.at[idx_slice], gbufs[ph], g_sems[ph]
            )

        fire_gather(0, 0)
        fire_gather(1, 1)
        pe_copy.wait()

        def compute(gbuf, obuf, c):
            def row_body(r, _):
                pe_off = (c * CHUNK + r) * PE_W
                PF = 2
                ws, gas, gbs = [], [], []
                for j in range(PAIRS + PF):
                    if j < PAIRS:
                        col = j * 2 * L
                        ws.append(pe_v[pl.ds(pe_off + j * L, L)])
                        gas.append(gbuf[r, pl.ds(col, L)])
                        gbs.append(gbuf[r, pl.ds(col + L, L)])
                    if j >= PF:
                        kk = j - PF
                        colk = kk * 2 * L
                        w = ws[kk]
                        pa = lax.bitcast_convert_type(w << 16, jnp.float32)
                        pb = lax.bitcast_convert_type(
                            (w >> 16) << 16, jnp.float32)
                        obuf[r, pl.ds(colk, L)] = gas[kk] * SCALE + pa
                        obuf[r, pl.ds(colk + L, L)] = gbs[kk] * SCALE + pb
                return 0

            lax.fori_loop(0, CHUNK, row_body, 0)

        def outer(o, _):
            for ph in range(2):
                s = 2 * o + ph
                b = s // STEPS
                c = s % STEPS
                gbuf, obuf = gbufs[ph], obufs[ph]
                pltpu.make_async_copy(
                    table_hbm.at[pl.ds(0, CHUNK)], gbuf, g_sems[ph]
                ).wait()

                @pl.when(o >= 1)
                def _():
                    pltpu.make_async_copy(
                        obuf, out_hbm.at[0, pl.ds(0, CHUNK), :], st_sems[ph]
                    ).wait()

                compute(gbuf, obuf, c)
                pltpu.async_copy(
                    obuf,
                    out_hbm.at[b, pl.ds(s0 + c * CHUNK, CHUNK), :],
                    st_sems[ph],
                )

                @pl.when(s < NSTEP - 2)
                def _():
                    fire_gather(s + 2, ph)

            return 0

        lax.fori_loop(0, NSTEP // 2, outer, 0)

        for ph in range(2):
            pltpu.make_async_copy(
                obufs[ph], out_hbm.at[0, pl.ds(0, CHUNK), :], st_sems[ph]
            ).wait()

    return k(x, pe, table)


def kernel(x, table):
    pe = _positional_encoding_packed()
    return _sc_embed(x.reshape(-1).astype(jnp.int32), pe, table)


# speedup vs baseline: 1.0087x; 1.0004x over previous
import functools

import jax
import jax.numpy as jnp
import numpy as np
from jax import lax
from jax.experimental import pallas as pl
from jax.experimental.pallas import tpu as pltpu
from jax.experimental.pallas import tpu_sc as plsc

D_MODEL = 1024
MAX_LEN = 2048
BATCH = 4
SEQ = 2048
SCALE = 32.0

L = 16
NC = 2
NS = 16
NW = NC * NS
S_PER_W = SEQ // NW
CHUNK = 16
STEPS = S_PER_W // CHUNK
NSTEP = BATCH * STEPS
PAIRS = D_MODEL // (2 * L)
PE_W = D_MODEL // 2


def _positional_encoding_packed():
    d = D_MODEL / 2
    pos = np.arange(MAX_LEN)[:, np.newaxis]
    dims = np.arange(d)[np.newaxis, :] / d
    frequency = pos * (1.0 / 10000 ** dims)
    pe = np.concatenate([np.sin(frequency), np.cos(frequency)], axis=-1)
    pe = pe.astype(np.float32).reshape(MAX_LEN, D_MODEL // 32, 2, 16)
    u = pe.view(np.uint32)
    bits = ((u + 0x7FFF + ((u >> 16) & 1)) >> 16).astype(np.uint32)
    words = bits[:, :, 0, :] | (bits[:, :, 1, :] << 16)
    words = words.astype(np.uint32).view(np.int32)
    return jnp.asarray(words.reshape(MAX_LEN * PE_W))


def _sc_embed(x, pe, table):
    mesh = plsc.VectorSubcoreMesh(core_axis_name="c", subcore_axis_name="s")

    @functools.partial(
        pl.kernel,
        mesh=mesh,
        out_type=jax.ShapeDtypeStruct((BATCH, SEQ, D_MODEL), jnp.float32),
        scratch_types=[
            pltpu.VMEM((BATCH * S_PER_W,), jnp.int32),
            pltpu.VMEM((S_PER_W * PE_W,), jnp.int32),
            pltpu.VMEM((CHUNK, D_MODEL), jnp.float32),
            pltpu.VMEM((CHUNK, D_MODEL), jnp.float32),
            pltpu.VMEM((CHUNK, D_MODEL), jnp.float32),
            pltpu.VMEM((CHUNK, D_MODEL), jnp.float32),
            pltpu.SemaphoreType.DMA,
            pltpu.SemaphoreType.DMA,
            pltpu.SemaphoreType.DMA,
            pltpu.SemaphoreType.DMA,
            pltpu.SemaphoreType.DMA,
            pltpu.SemaphoreType.DMA((BATCH,)),
        ],
    )
    def k(x_hbm, pe_hbm, table_hbm, out_hbm, idx_v, pe_v,
          gbuf0, gbuf1, obuf0, obuf1, g_sem0, g_sem1, st_sem0, st_sem1,
          pe_sem, idx_sems):
        gbufs = (gbuf0, gbuf1)
        obufs = (obuf0, obuf1)
        g_sems = (g_sem0, g_sem1)
        st_sems = (st_sem0, st_sem1)

        wid = lax.axis_index("s") * NC + lax.axis_index("c")
        s0 = wid * S_PER_W

        pe_copy = pltpu.async_copy(
            pe_hbm.at[pl.ds(s0 * PE_W, S_PER_W * PE_W)], pe_v, pe_sem
        )
        idx_copies = [
            pltpu.async_copy(
                x_hbm.at[pl.ds(b * SEQ + s0, S_PER_W)],
                idx_v.at[pl.ds(b * S_PER_W, S_PER_W)],
                idx_sems.at[b],
            )
            for b in range(BATCH)
        ]
        for cp in idx_copies:
            cp.wait()

        def fire_gather(s, ph):
            idx_slice = idx_v.at[pl.ds(s * CHUNK, CHUNK)]
            return pltpu.async_copy(
                table_hbm.at[idx_slice], gbufs[ph], g_sems[ph]
            )

        fire_gather(0, 0)
        fire_gather(1, 1)
        pe_copy.wait()

        def compute(gbuf, obuf, c):
            def row_body(r, _):
                pe_off = (c * CHUNK + r) * PE_W
                PF = 2
                ws, gas, gbs = [], [], []
                for j in range(PAIRS + PF):
                    if j < PAIRS:
                        col = j * 2 * L
                        ws.append(pe_v[pl.ds(pe_off + j * L, L)])
                        gas.append(gbuf[r, pl.ds(col, L)])
                        gbs.append(gbuf[r, pl.ds(col + L, L)])
                    if j >= PF:
                        kk = j - PF
                        colk = kk * 2 * L
                        w = ws[kk]
                        pa = lax.bitcast_convert_type(w << 16, jnp.float32)
                        pb = lax.bitcast_convert_type(
                            (w >> 16) << 16, jnp.float32)
                        obuf[r, pl.ds(colk, L)] = gas[kk] * SCALE + pa
                        obuf[r, pl.ds(colk + L, L)] = gbs[kk] * SCALE + pb
                return 0

            lax.fori_loop(0, CHUNK, row_body, 0)

        def outer(o, _):
            for ph in range(2):
                s = 2 * o + ph
                b = s // STEPS
                c = s % STEPS
                gbuf, obuf = gbufs[ph], obufs[ph]
                pltpu.make_async_copy(
                    table_hbm.at[pl.ds(0, CHUNK)], gbuf, g_sems[ph]
                ).wait()

                @pl.when(o >= 1)
                def _():
                    pltpu.make_async_copy(
                        obuf, out_hbm.at[0, pl.ds(0, CHUNK), :], st_sems[ph]
                    ).wait()

                compute(gbuf, obuf, c)
                pltpu.async_copy(
                    obuf,
                    out_hbm.at[b, pl.ds(s0 + c * CHUNK, CHUNK), :],
                    st_sems[ph],
                )

                @pl.when(s < NSTEP - 2)
                def _():
                    fire_gather(s + 2, ph)

            return 0

        lax.fori_loop(0, NSTEP // 2, outer, 0)

        for ph in range(2):
            pltpu.make_async_copy(
                obufs[ph], out_hbm.at[0, pl.ds(0, CHUNK), :], st_sems[ph]
            ).wait()

    return k(x, pe, table)


def kernel(x, table):
    pe = _positional_encoding_packed()
    return _sc_embed(x.reshape(-1).astype(jnp.int32), pe, table)


# speedup vs baseline: 1.0101x; 1.0014x over previous
import functools

import jax
import jax.numpy as jnp
import numpy as np
from jax import lax
from jax.experimental import pallas as pl
from jax.experimental.pallas import tpu as pltpu
from jax.experimental.pallas import tpu_sc as plsc

D_MODEL = 1024
MAX_LEN = 2048
BATCH = 4
SEQ = 2048
SCALE = 32.0

L = 16
NC = 2
NS = 16
NW = NC * NS
S_PER_W = SEQ // NW
CHUNK = 16
STEPS = S_PER_W // CHUNK
NSTEP = BATCH * STEPS
PAIRS = D_MODEL // (2 * L)
PE_W = D_MODEL // 2


def _positional_encoding_packed():
    d = D_MODEL / 2
    pos = np.arange(MAX_LEN)[:, np.newaxis]
    dims = np.arange(d)[np.newaxis, :] / d
    frequency = pos * (1.0 / 10000 ** dims)
    pe = np.concatenate([np.sin(frequency), np.cos(frequency)], axis=-1)
    pe = pe.astype(np.float32).reshape(MAX_LEN, D_MODEL // 32, 2, 16)
    u = pe.view(np.uint32)
    bits = ((u + 0x7FFF + ((u >> 16) & 1)) >> 16).astype(np.uint32)
    words = bits[:, :, 0, :] | (bits[:, :, 1, :] << 16)
    words = words.astype(np.uint32).view(np.int32)
    return jnp.asarray(words.reshape(MAX_LEN * PE_W))


def _sc_embed(x, pe, table):
    mesh = plsc.VectorSubcoreMesh(core_axis_name="c", subcore_axis_name="s")

    @functools.partial(
        pl.kernel,
        mesh=mesh,
        out_type=jax.ShapeDtypeStruct((BATCH, SEQ, D_MODEL), jnp.float32),
        scratch_types=[
            pltpu.VMEM((BATCH * S_PER_W,), jnp.int32),
            pltpu.VMEM((S_PER_W * PE_W,), jnp.int32),
            pltpu.VMEM((CHUNK, D_MODEL), jnp.float32),
            pltpu.VMEM((CHUNK, D_MODEL), jnp.float32),
            pltpu.VMEM((CHUNK, D_MODEL), jnp.float32),
            pltpu.VMEM((CHUNK, D_MODEL), jnp.float32),
            pltpu.SemaphoreType.DMA,
            pltpu.SemaphoreType.DMA,
            pltpu.SemaphoreType.DMA,
            pltpu.SemaphoreType.DMA,
            pltpu.SemaphoreType.DMA,
            pltpu.SemaphoreType.DMA((BATCH,)),
        ],
    )
    def k(x_hbm, pe_hbm, table_hbm, out_hbm, idx_v, pe_v,
          gbuf0, gbuf1, obuf0, obuf1, g_sem0, g_sem1, st_sem0, st_sem1,
          pe_sem, idx_sems):
        gbufs = (gbuf0, gbuf1)
        obufs = (obuf0, obuf1)
        g_sems = (g_sem0, g_sem1)
        st_sems = (st_sem0, st_sem1)

        wid = lax.axis_index("s") * NC + lax.axis_index("c")
        s0 = wid * S_PER_W

        pe_copy = pltpu.async_copy(
            pe_hbm.at[pl.ds(s0 * PE_W, S_PER_W * PE_W)], pe_v, pe_sem
        )
        idx_copies = [
            pltpu.async_copy(
                x_hbm.at[pl.ds(b * SEQ + s0, S_PER_W)],
                idx_v.at[pl.ds(b * S_PER_W, S_PER_W)],
                idx_sems.at[b],
            )
            for b in range(BATCH)
        ]
        for cp in idx_copies:
            cp.wait()

        def fire_gather(s, ph):
            idx_slice = idx_v.at[pl.ds(s * CHUNK, CHUNK)]
            return pltpu.async_copy(
                table_hbm.at[idx_slice], gbufs[ph], g_sems[ph]
            )

        fire_gather(0, 0)
        fire_gather(1, 1)
        pe_copy.wait()

        def compute(gbuf, obuf, c):
            def row_body(r, _):
                pe_off = (c * CHUNK + r) * PE_W
                PF = 2
                ws, gas, gbs = [], [], []
                for j in range(PAIRS + PF):
                    if j < PAIRS:
                        col = j * 2 * L
                        ws.append(pe_v[pl.ds(pe_off + j * L, L)])
                        gas.append(gbuf[r, pl.ds(col, L)])
                        gbs.append(gbuf[r, pl.ds(col + L, L)])
                    if j >= PF:
                        kk = j - PF
                        colk = kk * 2 * L
                        w = ws[kk]
                        pa = lax.bitcast_convert_type(w << 16, jnp.float32)
                        pb = lax.bitcast_convert_type(
                            w & jnp.int32(-65536), jnp.float32)
                        obuf[r, pl.ds(colk, L)] = gas[kk] * SCALE + pa
                        obuf[r, pl.ds(colk + L, L)] = gbs[kk] * SCALE + pb
                return 0

            lax.fori_loop(0, CHUNK, row_body, 0)

        def outer(o, _):
            for ph in range(2):
                s = 2 * o + ph
                b = s // STEPS
                c = s % STEPS
                gbuf, obuf = gbufs[ph], obufs[ph]
                pltpu.make_async_copy(
                    table_hbm.at[pl.ds(0, CHUNK)], gbuf, g_sems[ph]
                ).wait()

                @pl.when(o >= 1)
                def _():
                    pltpu.make_async_copy(
                        obuf, out_hbm.at[0, pl.ds(0, CHUNK), :], st_sems[ph]
                    ).wait()

                compute(gbuf, obuf, c)
                pltpu.async_copy(
                    obuf,
                    out_hbm.at[b, pl.ds(s0 + c * CHUNK, CHUNK), :],
                    st_sems[ph],
                )

                @pl.when(s < NSTEP - 2)
                def _():
                    fire_gather(s + 2, ph)

            return 0

        lax.fori_loop(0, NSTEP // 2, outer, 0)

        for ph in range(2):
            pltpu.make_async_copy(
                obufs[ph], out_hbm.at[0, pl.ds(0, CHUNK), :], st_sems[ph]
            ).wait()

    return k(x, pe, table)


def kernel(x, table):
    pe = _positional_encoding_packed()
    return _sc_embed(x.reshape(-1).astype(jnp.int32), pe, table)


# speedup vs baseline: 1.0299x; 1.0197x over previous
import functools

import jax
import jax.numpy as jnp
import numpy as np
from jax import lax
from jax.experimental import pallas as pl
from jax.experimental.pallas import tpu as pltpu
from jax.experimental.pallas import tpu_sc as plsc

D_MODEL = 1024
MAX_LEN = 2048
BATCH = 4
SEQ = 2048
SCALE = 32.0

L = 16
NC = 2
NS = 16
NW = NC * NS
S_PER_W = SEQ // NW
CHUNK = 16
STEPS = S_PER_W // CHUNK
NSTEP = BATCH * STEPS
PAIRS = D_MODEL // (2 * L)
PE_W = D_MODEL // 2


def _positional_encoding_packed():
    d = D_MODEL / 2
    pos = np.arange(MAX_LEN)[:, np.newaxis]
    dims = np.arange(d)[np.newaxis, :] / d
    frequency = pos * (1.0 / 10000 ** dims)
    pe = np.concatenate([np.sin(frequency), np.cos(frequency)], axis=-1)
    pe = pe.astype(np.float32).reshape(MAX_LEN, D_MODEL // 32, 2, 16)
    u = pe.view(np.uint32)
    bits = ((u + 0x7FFF + ((u >> 16) & 1)) >> 16).astype(np.uint32)
    words = bits[:, :, 0, :] | (bits[:, :, 1, :] << 16)
    words = words.astype(np.uint32).view(np.int32)
    return jnp.asarray(words.reshape(MAX_LEN * PE_W))


def _sc_embed(x, pe, table):
    mesh = plsc.VectorSubcoreMesh(core_axis_name="c", subcore_axis_name="s")

    @functools.partial(
        pl.kernel,
        mesh=mesh,
        out_type=jax.ShapeDtypeStruct((BATCH, SEQ, D_MODEL), jnp.float32),
        scratch_types=[
            pltpu.VMEM((BATCH * S_PER_W,), jnp.int32),
            pltpu.VMEM((S_PER_W * PE_W,), jnp.int32),
            pltpu.VMEM((CHUNK, D_MODEL), jnp.float32),
            pltpu.VMEM((CHUNK, D_MODEL), jnp.float32),
            pltpu.VMEM((CHUNK, D_MODEL), jnp.float32),
            pltpu.VMEM((CHUNK, D_MODEL), jnp.float32),
            pltpu.SemaphoreType.DMA,
            pltpu.SemaphoreType.DMA,
            pltpu.SemaphoreType.DMA,
            pltpu.SemaphoreType.DMA,
            pltpu.SemaphoreType.DMA,
            pltpu.SemaphoreType.DMA((BATCH,)),
        ],
    )
    def k(x_hbm, pe_hbm, table_hbm, out_hbm, idx_v, pe_v,
          gbuf0, gbuf1, obuf0, obuf1, g_sem0, g_sem1, st_sem0, st_sem1,
          pe_sem, idx_sems):
        gbufs = (gbuf0, gbuf1)
        obufs = (obuf0, obuf1)
        g_sems = (g_sem0, g_sem1)
        st_sems = (st_sem0, st_sem1)

        wid = lax.axis_index("s") * NC + lax.axis_index("c")
        s0 = wid * S_PER_W

        pe_copy = pltpu.async_copy(
            pe_hbm.at[pl.ds(s0 * PE_W, S_PER_W * PE_W)], pe_v, pe_sem
        )
        idx_copies = [
            pltpu.async_copy(
                x_hbm.at[pl.ds(b * SEQ + s0, S_PER_W)],
                idx_v.at[pl.ds(b * S_PER_W, S_PER_W)],
                idx_sems.at[b],
            )
            for b in range(BATCH)
        ]
        for cp in idx_copies:
            cp.wait()

        def fire_gather(s, ph):
            idx_slice = idx_v.at[pl.ds(s * CHUNK, CHUNK)]
            return pltpu.async_copy(
                table_hbm.at[idx_slice], gbufs[ph], g_sems[ph]
            )

        fire_gather(0, 0)
        fire_gather(1, 1)
        pe_copy.wait()

        def compute(gbuf, obuf, c):
            def row_body(r, _):
                pe_off = (c * CHUNK + r) * PE_W
                PF = 3
                ws, gas, gbs = [], [], []
                for j in range(PAIRS + PF):
                    if j < PAIRS:
                        col = j * 2 * L
                        ws.append(pe_v[pl.ds(pe_off + j * L, L)])
                        gas.append(gbuf[r, pl.ds(col, L)])
                        gbs.append(gbuf[r, pl.ds(col + L, L)])
                    if j >= PF:
                        kk = j - PF
                        colk = kk * 2 * L
                        w = ws[kk]
                        pa = lax.bitcast_convert_type(w << 16, jnp.float32)
                        pb = lax.bitcast_convert_type(
                            w & jnp.int32(-65536), jnp.float32)
                        obuf[r, pl.ds(colk, L)] = gas[kk] * SCALE + pa
                        obuf[r, pl.ds(colk + L, L)] = gbs[kk] * SCALE + pb
                return 0

            lax.fori_loop(0, CHUNK, row_body, 0)

        def outer(o, _):
            for ph in range(2):
                s = 2 * o + ph
                b = s // STEPS
                c = s % STEPS
                gbuf, obuf = gbufs[ph], obufs[ph]
                pltpu.make_async_copy(
                    table_hbm.at[pl.ds(0, CHUNK)], gbuf, g_sems[ph]
                ).wait()

                @pl.when(o >= 1)
                def _():
                    pltpu.make_async_copy(
                        obuf, out_hbm.at[0, pl.ds(0, CHUNK), :], st_sems[ph]
                    ).wait()

                compute(gbuf, obuf, c)
                pltpu.async_copy(
                    obuf,
                    out_hbm.at[b, pl.ds(s0 + c * CHUNK, CHUNK), :],
                    st_sems[ph],
                )

                @pl.when(s < NSTEP - 2)
                def _():
                    fire_gather(s + 2, ph)

            return 0

        lax.fori_loop(0, NSTEP // 2, outer, 0)

        for ph in range(2):
            pltpu.make_async_copy(
                obufs[ph], out_hbm.at[0, pl.ds(0, CHUNK), :], st_sems[ph]
            ).wait()

    return k(x, pe, table)


def kernel(x, table):
    pe = _positional_encoding_packed()
    return _sc_embed(x.reshape(-1).astype(jnp.int32), pe, table)


# speedup vs baseline: 1.0320x; 1.0020x over previous
import functools

import jax
import jax.numpy as jnp
import numpy as np
from jax import lax
from jax.experimental import pallas as pl
from jax.experimental.pallas import tpu as pltpu
from jax.experimental.pallas import tpu_sc as plsc

D_MODEL = 1024
MAX_LEN = 2048
BATCH = 4
SEQ = 2048
SCALE = 32.0

L = 16
NC = 2
NS = 16
NW = NC * NS
S_PER_W = SEQ // NW
CHUNK = 16
STEPS = S_PER_W // CHUNK
NSTEP = BATCH * STEPS
PAIRS = D_MODEL // (2 * L)
PE_W = D_MODEL // 2


def _positional_encoding_packed():
    d = D_MODEL / 2
    pos = np.arange(MAX_LEN)[:, np.newaxis]
    dims = np.arange(d)[np.newaxis, :] / d
    frequency = pos * (1.0 / 10000 ** dims)
    pe = np.concatenate([np.sin(frequency), np.cos(frequency)], axis=-1)
    pe = pe.astype(np.float32).reshape(MAX_LEN, D_MODEL // 32, 2, 16)
    u = pe.view(np.uint32)
    bits = ((u + 0x7FFF + ((u >> 16) & 1)) >> 16).astype(np.uint32)
    words = bits[:, :, 0, :] | (bits[:, :, 1, :] << 16)
    words = words.astype(np.uint32).view(np.int32)
    return jnp.asarray(words.reshape(MAX_LEN * PE_W))


def _sc_embed(x, pe, table):
    mesh = plsc.VectorSubcoreMesh(core_axis_name="c", subcore_axis_name="s")

    @functools.partial(
        pl.kernel,
        mesh=mesh,
        out_type=jax.ShapeDtypeStruct((BATCH, SEQ, D_MODEL), jnp.float32),
        scratch_types=[
            pltpu.VMEM((BATCH * S_PER_W,), jnp.int32),
            pltpu.VMEM((S_PER_W * PE_W,), jnp.int32),
            pltpu.VMEM((CHUNK, D_MODEL), jnp.float32),
            pltpu.VMEM((CHUNK, D_MODEL), jnp.float32),
            pltpu.VMEM((CHUNK, D_MODEL), jnp.float32),
            pltpu.VMEM((CHUNK, D_MODEL), jnp.float32),
            pltpu.SemaphoreType.DMA,
            pltpu.SemaphoreType.DMA,
            pltpu.SemaphoreType.DMA,
            pltpu.SemaphoreType.DMA,
            pltpu.SemaphoreType.DMA,
            pltpu.SemaphoreType.DMA((BATCH,)),
        ],
    )
    def k(x_hbm, pe_hbm, table_hbm, out_hbm, idx_v, pe_v,
          gbuf0, gbuf1, obuf0, obuf1, g_sem0, g_sem1, st_sem0, st_sem1,
          pe_sem, idx_sems):
        gbufs = (gbuf0, gbuf1)
        obufs = (obuf0, obuf1)
        g_sems = (g_sem0, g_sem1)
        st_sems = (st_sem0, st_sem1)

        wid = lax.axis_index("s") * NC + lax.axis_index("c")
        s0 = wid * S_PER_W

        pe_copy = pltpu.async_copy(
            pe_hbm.at[pl.ds(s0 * PE_W, S_PER_W * PE_W)], pe_v, pe_sem
        )
        idx_copies = [
            pltpu.async_copy(
                x_hbm.at[pl.ds(b * SEQ + s0, S_PER_W)],
                idx_v.at[pl.ds(b * S_PER_W, S_PER_W)],
                idx_sems.at[b],
            )
            for b in range(BATCH)
        ]
        for cp in idx_copies:
            cp.wait()

        def fire_gather(s, ph):
            idx_slice = idx_v.at[pl.ds(s * CHUNK, CHUNK)]
            return pltpu.async_copy(
                table_hbm.at[idx_slice], gbufs[ph], g_sems[ph]
            )

        fire_gather(0, 0)
        fire_gather(1, 1)
        pe_copy.wait()

        def compute(gbuf, obuf, c):
            def row_body(r, _):
                pe_off = (c * CHUNK + r) * PE_W
                PF = 4
                ws, gas, gbs = [], [], []
                for j in range(PAIRS + PF):
                    if j < PAIRS:
                        col = j * 2 * L
                        ws.append(pe_v[pl.ds(pe_off + j * L, L)])
                        gas.append(gbuf[r, pl.ds(col, L)])
                        gbs.append(gbuf[r, pl.ds(col + L, L)])
                    if j >= PF:
                        kk = j - PF
                        colk = kk * 2 * L
                        w = ws[kk]
                        pa = lax.bitcast_convert_type(w << 16, jnp.float32)
                        pb = lax.bitcast_convert_type(
                            w & jnp.int32(-65536), jnp.float32)
                        obuf[r, pl.ds(colk, L)] = gas[kk] * SCALE + pa
                        obuf[r, pl.ds(colk + L, L)] = gbs[kk] * SCALE + pb
                return 0

            lax.fori_loop(0, CHUNK, row_body, 0)

        def outer(o, _):
            for ph in range(2):
                s = 2 * o + ph
                b = s // STEPS
                c = s % STEPS
                gbuf, obuf = gbufs[ph], obufs[ph]
                pltpu.make_async_copy(
                    table_hbm.at[pl.ds(0, CHUNK)], gbuf, g_sems[ph]
                ).wait()

                @pl.when(o >= 1)
                def _():
                    pltpu.make_async_copy(
                        obuf, out_hbm.at[0, pl.ds(0, CHUNK), :], st_sems[ph]
                    ).wait()

                compute(gbuf, obuf, c)
                pltpu.async_copy(
                    obuf,
                    out_hbm.at[b, pl.ds(s0 + c * CHUNK, CHUNK), :],
                    st_sems[ph],
                )

                @pl.when(s < NSTEP - 2)
                def _():
                    fire_gather(s + 2, ph)

            return 0

        lax.fori_loop(0, NSTEP // 2, outer, 0)

        for ph in range(2):
            pltpu.make_async_copy(
                obufs[ph], out_hbm.at[0, pl.ds(0, CHUNK), :], st_sems[ph]
            ).wait()

    return k(x, pe, table)


def kernel(x, table):
    pe = _positional_encoding_packed()
    return _sc_embed(x.reshape(-1).astype(jnp.int32), pe, table)
